# baseline (speedup 1.0000x reference)
import functools

import jax
import jax.numpy as jnp
from jax.experimental import pallas as pl
from jax.experimental.pallas import tpu as pltpu


def _permuter_kernel(zt_row_ref, zt_col_ref, out_ref, buf_ref, sems, *,
                     ck, ns):
    b = pl.program_id(0)
    nb = pl.num_programs(0)
    n = zt_row_ref.shape[2]
    g = n // ns

    vj = zt_row_ref[0]
    vcol = zt_col_ref[0]
    jidx = jax.lax.broadcasted_iota(jnp.int32, (1, n), 1)
    acc = jnp.zeros((1, n), dtype=jnp.int32)
    for c in range(n // ck):
        vk = vcol[c * ck:(c + 1) * ck, :]
        kidx = c * ck + jax.lax.broadcasted_iota(jnp.int32, (ck, 1), 0)
        smaller = (vk < vj) | ((vk == vj) & (kidx < jidx))
        acc = acc + jnp.sum(smaller.astype(jnp.int32), axis=0,
                            keepdims=True)
    rank = jnp.broadcast_to(acc, (g, n))

    for s in range(ns):
        @pl.when(b > 0)
        def _wait_prev():
            pltpu.make_async_copy(
                buf_ref.at[s], out_ref.at[b, :, s, :], sems.at[s]).wait()
        ii = s + ns * jax.lax.broadcasted_iota(jnp.int32, (g, n), 0)
        buf_ref[s] = (rank == ii).astype(jnp.float32)
        pltpu.make_async_copy(
            buf_ref.at[s], out_ref.at[b, :, s, :], sems.at[s]).start()

    @pl.when(b == nb - 1)
    def _drain():
        for s in range(ns):
            pltpu.make_async_copy(
                buf_ref.at[s], out_ref.at[b, :, s, :], sems.at[s]).wait()


@jax.jit
def kernel(z, tau):
    B, N = z.shape
    eps = jnp.finfo(z.dtype).eps
    u = jax.random.uniform(jax.random.key(42), z.shape, dtype=z.dtype)
    g = -jnp.log(-jnp.log(u + eps) + eps)
    zt = z + tau * g

    CK = 256
    NS = 8

    zt_row = zt.reshape(B, 1, N)
    zt_col = zt.reshape(B, N, 1)

    out = pl.pallas_call(
        functools.partial(_permuter_kernel, ck=CK, ns=NS),
        grid=(B,),
        in_specs=[
            pl.BlockSpec((1, 1, N), lambda b: (b, 0, 0)),
            pl.BlockSpec((1, N, 1), lambda b: (b, 0, 0)),
        ],
        out_specs=pl.BlockSpec(memory_space=pl.ANY),
        out_shape=jax.ShapeDtypeStruct((B, N // NS, NS, N), z.dtype),
        scratch_shapes=[
            pltpu.VMEM((NS, N // NS, N), jnp.float32),
            pltpu.SemaphoreType.DMA((NS,)),
        ],
    )(zt_row, zt_col)
    return out.reshape(B, N, N)
